# baseline (speedup 1.0000x reference)
import jax
import jax.numpy as jnp
from jax import lax
from jax.experimental import pallas as pl
from jax.experimental.pallas import tpu as pltpu
from jax.experimental.pallas import tpu_sc as plsc

_SEQ = 8192
_NS = 16
_L = 16
_FLAT = 2 * _SEQ
_BLK = _FLAT // _NS
_PAD = _L
_UNROLL = 8

_POS = 5316


def _lane_sum(v, buf):
    for sh in (8, 4, 2, 1):
        buf[pl.ds(0, _L)] = v
        buf[pl.ds(_L, _L)] = v
        v = v + buf[pl.ds(sh, _L)]
    return v


def _make_body(pos):
    owner = (2 * pos) // _BLK
    ochunk = ((2 * pos) % _BLK) // _L
    olane = (2 * pos) % _L

    def _gate_body(ggs_hbm, out_hbm, in_v, out_v, loc_v, shared_v, red_v,
                   buf_v):
        s = lax.axis_index("s")
        base = s * _BLK
        one = jnp.full((_L,), 1.0, jnp.float32)
        zero = jnp.full((_L,), 0.0, jnp.float32)
        lane = lax.iota(jnp.int32, _L)
        even = (lane % 2) == 0

        pltpu.sync_copy(ggs_hbm.at[pl.ds(base, _BLK)], in_v.at[pl.ds(_PAD, _BLK)])

        def step(j, acc):
            for u in range(_UNROLL):
                off = (j * _UNROLL + u) * _L
                v = in_v[pl.ds(_PAD + off, _L)]
                sl = in_v[pl.ds(_PAD + off + 1, _L)]
                sr = in_v[pl.ds(_PAD + off - 1, _L)]
                cmp = v - jnp.where(even, sl, sr)
                sel = (cmp > zero) | (even & (cmp == zero))
                o = jnp.where(sel, one, zero)
                out_v[pl.ds(off, _L)] = o
                acc = acc + jnp.where(even, o, zero)
            return acc

        acc = lax.fori_loop(0, _BLK // (_L * _UNROLL), step, zero)

        loc_v[...] = acc
        pltpu.sync_copy(loc_v, shared_v.at[s])
        pltpu.sync_copy(out_v, out_hbm.at[pl.ds(base, _BLK)])
        plsc.subcore_barrier()

        @pl.when(s == owner)
        def _():
            pltpu.sync_copy(shared_v, red_v)
            tot = zero
            for t in range(_NS):
                tot = tot + red_v[t]
            tot = _lane_sum(tot, buf_v)
            n_rows = jnp.full((_L,), float(_SEQ), jnp.float32)
            cond = (tot == n_rows) | (tot == zero)
            pair = (lane == olane) | (lane == olane + 1)
            o = out_v[pl.ds(ochunk * _L, _L)]
            o = jnp.where(cond & pair, one - o, o)
            out_v[pl.ds(ochunk * _L, _L)] = o
            pltpu.sync_copy(out_v.at[pl.ds(ochunk * _L, _L)],
                            out_hbm.at[pl.ds(base + ochunk * _L, _L)])

    return _gate_body


@jax.jit
def _gate_sc(ggs_flat):
    mesh = plsc.VectorSubcoreMesh(core_axis_name="c", subcore_axis_name="s",
                                  num_cores=1, num_subcores=_NS)
    f32 = jnp.float32
    run = pl.kernel(
        _make_body(_POS),
        out_type=jax.ShapeDtypeStruct((_FLAT,), f32),
        mesh=mesh,
        scratch_types=[
            pltpu.VMEM((_BLK + 2 * _PAD,), f32),
            pltpu.VMEM((_BLK,), f32),
            pltpu.VMEM((_L,), f32),
            pltpu.VMEM_SHARED((_NS, _L), f32),
            pltpu.VMEM((_NS, _L), f32),
            pltpu.VMEM((2 * _L,), f32),
        ],
        name="me_gated_balance_mask",
    )
    return run(ggs_flat)


def kernel(x, W, global_gate_score):
    del x, W
    flat = _gate_sc(global_gate_score.reshape(-1))
    out = flat.reshape(_SEQ, 2)
    return (out[:, 0], out[:, 1])


# speedup vs baseline: 1.5434x; 1.5434x over previous
import jax
import jax.numpy as jnp
from jax import lax
from jax.experimental import pallas as pl
from jax.experimental.pallas import tpu as pltpu
from jax.experimental.pallas import tpu_sc as plsc

_SEQ = 8192
_NS = 16
_L = 16
_ROWS = _SEQ // _NS
_UNROLL = 8

_POS = 5316


def _lane_sum(v, buf):
    for sh in (8, 4, 2, 1):
        buf[pl.ds(0, _L)] = v
        buf[pl.ds(_L, _L)] = v
        v = v + buf[pl.ds(sh, _L)]
    return v


def _make_body(pos):
    owner = pos // _ROWS
    ochunk = (pos % _ROWS) // _L
    olane = pos % _L

    def _gate_body(col0_hbm, col1_hbm, out0_hbm, out1_hbm,
                   a_v, b_v, o0_v, o1_v, loc_v, shared_v, red_v, buf_v):
        s = lax.axis_index("s")
        rbase = s * _ROWS
        one = jnp.full((_L,), 1.0, jnp.float32)
        zero = jnp.full((_L,), 0.0, jnp.float32)
        lane = lax.iota(jnp.int32, _L)

        pltpu.sync_copy(col0_hbm.at[pl.ds(rbase, _ROWS)], a_v)
        pltpu.sync_copy(col1_hbm.at[pl.ds(rbase, _ROWS)], b_v)

        def step(j, acc):
            for u in range(_UNROLL):
                off = (j * _UNROLL + u) * _L
                a = a_v[pl.ds(off, _L)]
                b = b_v[pl.ds(off, _L)]
                h = jnp.where(a >= b, one, zero)
                o0_v[pl.ds(off, _L)] = h
                o1_v[pl.ds(off, _L)] = one - h
                acc = acc + h
            return acc

        acc = lax.fori_loop(0, _ROWS // (_L * _UNROLL), step, zero)

        loc_v[...] = acc
        pltpu.sync_copy(loc_v, shared_v.at[s])
        pltpu.sync_copy(o0_v, out0_hbm.at[pl.ds(rbase, _ROWS)])
        pltpu.sync_copy(o1_v, out1_hbm.at[pl.ds(rbase, _ROWS)])
        plsc.subcore_barrier()

        @pl.when(s == owner)
        def _():
            pltpu.sync_copy(shared_v, red_v)
            tot = zero
            for t in range(_NS):
                tot = tot + red_v[t]
            tot = _lane_sum(tot, buf_v)
            n_rows = jnp.full((_L,), float(_SEQ), jnp.float32)
            cond = (tot == n_rows) | (tot == zero)
            hit = cond & (lane == olane)
            a = o0_v[pl.ds(ochunk * _L, _L)]
            b = o1_v[pl.ds(ochunk * _L, _L)]
            o0_v[pl.ds(ochunk * _L, _L)] = jnp.where(hit, b, a)
            o1_v[pl.ds(ochunk * _L, _L)] = jnp.where(hit, a, b)
            off = ochunk * _L
            pltpu.sync_copy(o0_v.at[pl.ds(off, _L)],
                            out0_hbm.at[pl.ds(rbase + off, _L)])
            pltpu.sync_copy(o1_v.at[pl.ds(off, _L)],
                            out1_hbm.at[pl.ds(rbase + off, _L)])

    return _gate_body


@jax.jit
def _gate_sc(col0, col1):
    mesh = plsc.VectorSubcoreMesh(core_axis_name="c", subcore_axis_name="s",
                                  num_cores=1, num_subcores=_NS)
    f32 = jnp.float32
    run = pl.kernel(
        _make_body(_POS),
        out_type=(jax.ShapeDtypeStruct((_SEQ,), f32),
                  jax.ShapeDtypeStruct((_SEQ,), f32)),
        mesh=mesh,
        scratch_types=[
            pltpu.VMEM((_ROWS,), f32),
            pltpu.VMEM((_ROWS,), f32),
            pltpu.VMEM((_ROWS,), f32),
            pltpu.VMEM((_ROWS,), f32),
            pltpu.VMEM((_L,), f32),
            pltpu.VMEM_SHARED((_NS, _L), f32),
            pltpu.VMEM((_NS, _L), f32),
            pltpu.VMEM((2 * _L,), f32),
        ],
        name="me_gated_balance_mask",
    )
    return run(col0, col1)


def kernel(x, W, global_gate_score):
    del x, W
    ggs = global_gate_score
    return _gate_sc(ggs[:, 0], ggs[:, 1])


# speedup vs baseline: 1.5865x; 1.0280x over previous
import jax
import jax.numpy as jnp
from jax import lax
from jax.experimental import pallas as pl
from jax.experimental.pallas import tpu as pltpu
from jax.experimental.pallas import tpu_sc as plsc

_SEQ = 8192
_NS = 16
_L = 16
_ROWS = _SEQ // _NS
_UNROLL = 8

_POS = 5316


def _lane_sum(v, buf):
    for sh in (8, 4, 2, 1):
        buf[pl.ds(0, _L)] = v
        buf[pl.ds(_L, _L)] = v
        v = v + buf[pl.ds(sh, _L)]
    return v


def _make_body(pos):
    owner = pos // _ROWS
    ochunk = (pos % _ROWS) // _L
    olane = pos % _L

    def _gate_body(col0_hbm, col1_hbm, out0_hbm, out1_hbm,
                   a_v, b_v, o0_v, o1_v, loc_v, shared_v, red_v, buf_v, sem):
        s = lax.axis_index("s")
        rbase = s * _ROWS
        one = jnp.full((_L,), 1.0, jnp.float32)
        zero = jnp.full((_L,), 0.0, jnp.float32)
        lane = lax.iota(jnp.int32, _L)

        cp_a = pltpu.async_copy(col0_hbm.at[pl.ds(rbase, _ROWS)], a_v, sem)
        cp_b = pltpu.async_copy(col1_hbm.at[pl.ds(rbase, _ROWS)], b_v, sem)
        cp_a.wait()
        cp_b.wait()

        def step(j, acc):
            for u in range(_UNROLL):
                off = (j * _UNROLL + u) * _L
                a = a_v[pl.ds(off, _L)]
                b = b_v[pl.ds(off, _L)]
                h = jnp.where(a >= b, one, zero)
                o0_v[pl.ds(off, _L)] = h
                o1_v[pl.ds(off, _L)] = one - h
                acc = acc + h
            return acc

        acc = lax.fori_loop(0, _ROWS // (_L * _UNROLL), step, zero)

        cp0 = pltpu.async_copy(o0_v, out0_hbm.at[pl.ds(rbase, _ROWS)], sem)
        cp1 = pltpu.async_copy(o1_v, out1_hbm.at[pl.ds(rbase, _ROWS)], sem)
        loc_v[...] = _lane_sum(acc, buf_v)
        pltpu.sync_copy(loc_v, shared_v.at[s])
        cp0.wait()
        cp1.wait()
        plsc.subcore_barrier()

        @pl.when(s == owner)
        def _():
            pltpu.sync_copy(shared_v, red_v)
            tot = zero
            for t in range(_NS):
                tot = tot + red_v[t]
            n_rows = jnp.full((_L,), float(_SEQ), jnp.float32)
            cond = (tot == n_rows) | (tot == zero)
            hit = cond & (lane == olane)
            a = o0_v[pl.ds(ochunk * _L, _L)]
            b = o1_v[pl.ds(ochunk * _L, _L)]
            o0_v[pl.ds(ochunk * _L, _L)] = jnp.where(hit, b, a)
            o1_v[pl.ds(ochunk * _L, _L)] = jnp.where(hit, a, b)
            off = ochunk * _L
            pltpu.sync_copy(o0_v.at[pl.ds(off, _L)],
                            out0_hbm.at[pl.ds(rbase + off, _L)])
            pltpu.sync_copy(o1_v.at[pl.ds(off, _L)],
                            out1_hbm.at[pl.ds(rbase + off, _L)])

    return _gate_body


@jax.jit
def _gate_sc(col0, col1):
    mesh = plsc.VectorSubcoreMesh(core_axis_name="c", subcore_axis_name="s",
                                  num_cores=1, num_subcores=_NS)
    f32 = jnp.float32
    run = pl.kernel(
        _make_body(_POS),
        out_type=(jax.ShapeDtypeStruct((_SEQ,), f32),
                  jax.ShapeDtypeStruct((_SEQ,), f32)),
        mesh=mesh,
        scratch_types=[
            pltpu.VMEM((_ROWS,), f32),
            pltpu.VMEM((_ROWS,), f32),
            pltpu.VMEM((_ROWS,), f32),
            pltpu.VMEM((_ROWS,), f32),
            pltpu.VMEM((_L,), f32),
            pltpu.VMEM_SHARED((_NS, _L), f32),
            pltpu.VMEM((_NS, _L), f32),
            pltpu.VMEM((2 * _L,), f32),
            pltpu.SemaphoreType.DMA,
        ],
        name="me_gated_balance_mask",
    )
    return run(col0, col1)


def kernel(x, W, global_gate_score):
    del x, W
    ggs = global_gate_score
    return _gate_sc(ggs[:, 0], ggs[:, 1])


# speedup vs baseline: 1.5959x; 1.0059x over previous
import jax
import jax.numpy as jnp
from jax import lax
from jax.experimental import pallas as pl
from jax.experimental.pallas import tpu as pltpu
from jax.experimental.pallas import tpu_sc as plsc

_SEQ = 8192
_NS = 16
_L = 16
_ROWS = _SEQ // _NS
_UNROLL = 8

_POS = 5316


def _lane_sum(v, buf):
    for sh in (8, 4, 2, 1):
        buf[pl.ds(0, _L)] = v
        buf[pl.ds(_L, _L)] = v
        v = v + buf[pl.ds(sh, _L)]
    return v


def _make_body(pos):
    owner = pos // _ROWS
    ochunk = (pos % _ROWS) // _L
    olane = pos % _L

    def _gate_body(col0_hbm, col1_hbm, out0_hbm, out1_hbm,
                   a_v, b_v, o0_v, o1_v, loc_v, shared_v, red_v, buf_v, sem):
        s = lax.axis_index("s")
        rbase = s * _ROWS
        one = jnp.full((_L,), 1.0, jnp.float32)
        zero = jnp.full((_L,), 0.0, jnp.float32)
        lane = lax.iota(jnp.int32, _L)

        cp_a = pltpu.async_copy(col0_hbm.at[pl.ds(rbase, _ROWS)], a_v, sem)
        cp_b = pltpu.async_copy(col1_hbm.at[pl.ds(rbase, _ROWS)], b_v, sem)
        cp_a.wait()
        cp_b.wait()

        def step(j, acc):
            for u in range(_UNROLL):
                off = (j * _UNROLL + u) * _L
                a = a_v[pl.ds(off, _L)]
                b = b_v[pl.ds(off, _L)]
                h = jnp.where(a >= b, one, zero)
                o0_v[pl.ds(off, _L)] = h
                o1_v[pl.ds(off, _L)] = one - h
                acc = acc + h
            return acc

        acc = lax.fori_loop(0, _ROWS // (_L * _UNROLL), step, zero)

        @pl.when(s != owner)
        def _():
            pltpu.async_copy(o0_v, out0_hbm.at[pl.ds(rbase, _ROWS)], sem)
            pltpu.async_copy(o1_v, out1_hbm.at[pl.ds(rbase, _ROWS)], sem)

        loc_v[...] = _lane_sum(acc, buf_v)
        pltpu.sync_copy(loc_v, shared_v.at[s])
        plsc.subcore_barrier()

        @pl.when(s != owner)
        def _():
            pltpu.make_async_copy(o0_v, out0_hbm.at[pl.ds(rbase, _ROWS)],
                                  sem).wait()
            pltpu.make_async_copy(o1_v, out1_hbm.at[pl.ds(rbase, _ROWS)],
                                  sem).wait()

        @pl.when(s == owner)
        def _():
            pltpu.sync_copy(shared_v, red_v)
            tot = zero
            for t in range(_NS):
                tot = tot + red_v[t]
            n_rows = jnp.full((_L,), float(_SEQ), jnp.float32)
            cond = (tot == n_rows) | (tot == zero)
            hit = cond & (lane == olane)
            a = o0_v[pl.ds(ochunk * _L, _L)]
            b = o1_v[pl.ds(ochunk * _L, _L)]
            o0_v[pl.ds(ochunk * _L, _L)] = jnp.where(hit, b, a)
            o1_v[pl.ds(ochunk * _L, _L)] = jnp.where(hit, a, b)
            cp0 = pltpu.async_copy(o0_v, out0_hbm.at[pl.ds(rbase, _ROWS)],
                                   sem)
            cp1 = pltpu.async_copy(o1_v, out1_hbm.at[pl.ds(rbase, _ROWS)],
                                   sem)
            cp0.wait()
            cp1.wait()

    return _gate_body


@jax.jit
def _gate_sc(col0, col1):
    mesh = plsc.VectorSubcoreMesh(core_axis_name="c", subcore_axis_name="s",
                                  num_cores=1, num_subcores=_NS)
    f32 = jnp.float32
    run = pl.kernel(
        _make_body(_POS),
        out_type=(jax.ShapeDtypeStruct((_SEQ,), f32),
                  jax.ShapeDtypeStruct((_SEQ,), f32)),
        mesh=mesh,
        scratch_types=[
            pltpu.VMEM((_ROWS,), f32),
            pltpu.VMEM((_ROWS,), f32),
            pltpu.VMEM((_ROWS,), f32),
            pltpu.VMEM((_ROWS,), f32),
            pltpu.VMEM((_L,), f32),
            pltpu.VMEM_SHARED((_NS, _L), f32),
            pltpu.VMEM((_NS, _L), f32),
            pltpu.VMEM((2 * _L,), f32),
            pltpu.SemaphoreType.DMA,
        ],
        name="me_gated_balance_mask",
    )
    return run(col0, col1)


def kernel(x, W, global_gate_score):
    del x, W
    ggs = global_gate_score
    return _gate_sc(ggs[:, 0], ggs[:, 1])
